# baseline (speedup 1.0000x reference)
import functools

import jax
import jax.numpy as jnp
from jax import lax
from jax.experimental import pallas as pl
from jax.experimental.pallas import tpu as pltpu
from jax.experimental.pallas import tpu_sc as plsc

_NC = 2
_NS = 16
_NW = _NC * _NS

_B = 80
_DW = 16


def _sc_segsum(rows, src3d, dst3d, zeros_rows, zeros_deg, ones_b, with_deg):
    n, w = rows.shape
    nchunks = src3d.shape[1]
    sub_stride = 8 * (n // (8 * _NS))
    sub_rows = n - sub_stride * (_NS - 1)

    mesh = plsc.VectorSubcoreMesh(core_axis_name="c", subcore_axis_name="s")

    out_type = [jax.ShapeDtypeStruct((_NC, n, w), jnp.float32)]
    if with_deg:
        out_type.append(jax.ShapeDtypeStruct((_NC, n, _DW), jnp.float32))

    scratch = [
        pltpu.VMEM((nchunks, _B), jnp.int32),
        pltpu.VMEM((nchunks, _B), jnp.int32),
        pltpu.VMEM((2, _B, w), jnp.float32),
        pltpu.VMEM((_B, _DW), jnp.float32),
        pltpu.SemaphoreType.DMA((2,)),
        pltpu.VMEM_SHARED((n, w), jnp.float32),
        pltpu.VMEM_SHARED((n, _DW), jnp.float32),
    ]

    def body(rows_hbm, src_hbm, dst_hbm, zr_hbm, zd_hbm, ones_hbm,
             *refs):
        if with_deg:
            acc_out, deg_out = refs[0], refs[1]
            scr = refs[2:]
        else:
            acc_out = refs[0]
            scr = refs[1:]
        src_v, dst_v, rows_v, ones_v, sem, acc_sh, deg_sh = scr

        c = lax.axis_index("c")
        s = lax.axis_index("s")
        wid = s * _NC + c

        r0 = s * sub_stride
        pltpu.sync_copy(zr_hbm.at[pl.ds(r0, sub_rows)],
                        acc_sh.at[pl.ds(r0, sub_rows)])
        if with_deg:
            pltpu.sync_copy(zd_hbm.at[pl.ds(r0, sub_rows)],
                            deg_sh.at[pl.ds(r0, sub_rows)])
            pltpu.sync_copy(ones_hbm, ones_v)

        pltpu.sync_copy(src_hbm.at[wid], src_v)
        pltpu.sync_copy(dst_hbm.at[wid], dst_v)
        pltpu.async_copy(rows_hbm.at[src_v.at[0]], rows_v.at[0], sem.at[0])
        plsc.subcore_barrier()

        def step(j, carry):
            slot = lax.rem(j, 2)
            nslot = lax.rem(j + 1, 2)

            @pl.when(j + 1 < nchunks)
            def _():
                pltpu.async_copy(rows_hbm.at[src_v.at[j + 1]],
                                 rows_v.at[nslot], sem.at[nslot])

            pltpu.make_async_copy(rows_hbm.at[src_v.at[j]],
                                  rows_v.at[slot], sem.at[slot]).wait()
            pltpu.sync_copy(rows_v.at[slot], acc_sh.at[dst_v.at[j]], add=True)
            if with_deg:
                pltpu.sync_copy(ones_v, deg_sh.at[dst_v.at[j]], add=True)
            return carry

        lax.fori_loop(0, nchunks, step, 0)
        plsc.subcore_barrier()

        pltpu.sync_copy(acc_sh.at[pl.ds(r0, sub_rows)],
                        acc_out.at[c].at[pl.ds(r0, sub_rows)])
        if with_deg:
            pltpu.sync_copy(deg_sh.at[pl.ds(r0, sub_rows)],
                            deg_out.at[c].at[pl.ds(r0, sub_rows)])

    k = pl.kernel(body, out_type=tuple(out_type), mesh=mesh,
                  scratch_types=scratch,
                  compiler_params=pltpu.CompilerParams(
                      use_tc_tiling_on_sc=False))
    return k(rows, src3d, dst3d, zeros_rows, zeros_deg, ones_b)


def _sc_gather(z, ps3d, pd3d):
    n, w = z.shape
    nchunks = ps3d.shape[1]
    e_per_w = nchunks * _B
    e = _NW * e_per_w

    mesh = plsc.VectorSubcoreMesh(core_axis_name="c", subcore_axis_name="s")

    out_type = (jax.ShapeDtypeStruct((e, w), jnp.float32),
                jax.ShapeDtypeStruct((e, w), jnp.float32))
    scratch = [
        pltpu.VMEM((nchunks, _B), jnp.int32),
        pltpu.VMEM((nchunks, _B), jnp.int32),
        pltpu.VMEM((2, _B, w), jnp.float32),
        pltpu.VMEM((2, _B, w), jnp.float32),
        pltpu.SemaphoreType.DMA((2,)),
        pltpu.SemaphoreType.DMA((2,)),
    ]

    def body(z_hbm, ps_hbm, pd_hbm, zs_out, zd_out,
             ps_v, pd_v, rs_v, rd_v, sem_s, sem_d):
        c = lax.axis_index("c")
        s = lax.axis_index("s")
        wid = s * _NC + c
        base = wid * e_per_w

        pltpu.sync_copy(ps_hbm.at[wid], ps_v)
        pltpu.sync_copy(pd_hbm.at[wid], pd_v)
        pltpu.async_copy(z_hbm.at[ps_v.at[0]], rs_v.at[0], sem_s.at[0])
        pltpu.async_copy(z_hbm.at[pd_v.at[0]], rd_v.at[0], sem_d.at[0])

        def step(j, carry):
            slot = lax.rem(j, 2)
            nslot = lax.rem(j + 1, 2)

            @pl.when(j + 1 < nchunks)
            def _():
                pltpu.async_copy(z_hbm.at[ps_v.at[j + 1]],
                                 rs_v.at[nslot], sem_s.at[nslot])
                pltpu.async_copy(z_hbm.at[pd_v.at[j + 1]],
                                 rd_v.at[nslot], sem_d.at[nslot])

            pltpu.make_async_copy(z_hbm.at[ps_v.at[j]],
                                  rs_v.at[slot], sem_s.at[slot]).wait()
            pltpu.sync_copy(rs_v.at[slot], zs_out.at[pl.ds(base + j * _B, _B)])
            pltpu.make_async_copy(z_hbm.at[pd_v.at[j]],
                                  rd_v.at[slot], sem_d.at[slot]).wait()
            pltpu.sync_copy(rd_v.at[slot], zd_out.at[pl.ds(base + j * _B, _B)])
            return carry

        lax.fori_loop(0, nchunks, step, 0)

    k = pl.kernel(body, out_type=out_type, mesh=mesh, scratch_types=scratch,
                  compiler_params=pltpu.CompilerParams(
                      use_tc_tiling_on_sc=False))
    return k(z, ps3d, pd3d)


_BN = 2000
_BE = 2000


def _full(shape):
    return pl.BlockSpec(shape, lambda i: tuple(0 for _ in shape))


def _rows(shape):
    return pl.BlockSpec(shape, lambda i: (i,) + tuple(0 for _ in shape[1:]))


def _tc_in_proj(x, wl, wr):
    n, f = x.shape
    h = wl.shape[1]

    def body(x_ref, wl_ref, wr_ref, xl_ref, xr_ref):
        xb = x_ref[...]
        xl_ref[...] = jnp.dot(xb, wl_ref[...],
                              preferred_element_type=jnp.float32)
        xr_ref[...] = jnp.dot(xb, wr_ref[...],
                              preferred_element_type=jnp.float32)

    return pl.pallas_call(
        body,
        grid=(n // _BN,),
        in_specs=[_rows((_BN, f)), _full((f, h)), _full((f, h))],
        out_specs=[_rows((_BN, h)), _rows((_BN, h))],
        out_shape=(jax.ShapeDtypeStruct((n, h), jnp.float32),
                   jax.ShapeDtypeStruct((n, h), jnp.float32)),
    )(x, wl, wr)


def _tc_layer1(a0, a1, d0, d1, b1, xr, w2l, w2r):
    n, h = a0.shape
    d = w2l.shape[1]

    def body(a0_r, a1_r, d0_r, d1_r, b1_r, xr_r, w2l_r, w2r_r,
             hl_o, hr_o, deg_o):
        deg = jnp.maximum(d0_r[...][:, 0:1] + d1_r[...][:, 0:1], 1.0)
        mean = (a0_r[...] + a1_r[...]) / deg
        hcur = jnp.maximum(mean + b1_r[...] + xr_r[...], 0.0)
        hl_o[...] = jnp.dot(hcur, w2l_r[...],
                            preferred_element_type=jnp.float32)
        hr_o[...] = jnp.dot(hcur, w2r_r[...],
                            preferred_element_type=jnp.float32)
        deg_o[...] = deg

    return pl.pallas_call(
        body,
        grid=(n // _BN,),
        in_specs=[_rows((_BN, h)), _rows((_BN, h)),
                  _rows((_BN, _DW)), _rows((_BN, _DW)),
                  _full((1, h)), _rows((_BN, h)),
                  _full((h, d)), _full((h, d))],
        out_specs=[_rows((_BN, d)), _rows((_BN, d)), _rows((_BN, 1))],
        out_shape=(jax.ShapeDtypeStruct((n, d), jnp.float32),
                   jax.ShapeDtypeStruct((n, d), jnp.float32),
                   jax.ShapeDtypeStruct((n, 1), jnp.float32)),
    )(a0, a1, d0, d1, b1, xr, w2l, w2r)


def _tc_layer2(a0, a1, deg, b2, hr):
    n, d = a0.shape

    def body(a0_r, a1_r, deg_r, b2_r, hr_r, z_o):
        z_o[...] = (a0_r[...] + a1_r[...]) / deg_r[...] + b2_r[...] + hr_r[...]

    return pl.pallas_call(
        body,
        grid=(n // _BN,),
        in_specs=[_rows((_BN, d)), _rows((_BN, d)), _rows((_BN, 1)),
                  _full((1, d)), _rows((_BN, d))],
        out_specs=_rows((_BN, d)),
        out_shape=jax.ShapeDtypeStruct((n, d), jnp.float32),
    )(a0, a1, deg, b2, hr)


def _tc_predict(zs, zd, ts, wt1, bt1, wt2, bt2,
                wp1a, wp1b, wp1c, bp1, wp2, bp2, wp3, bp3):
    e, d = zs.shape
    t = wt2.shape[1]

    def body(zs_r, zd_r, ts_r, wt1_r, bt1_r, wt2_r, bt2_r,
             wp1a_r, wp1b_r, wp1c_r, bp1_r, wp2_r, bp2_r, wp3_r, bp3_r,
             out_o):
        tf = jnp.maximum(ts_r[...] * wt1_r[...] + bt1_r[...], 0.0)
        tf = jnp.dot(tf, wt2_r[...], preferred_element_type=jnp.float32)
        tf = tf + bt2_r[...]
        h1 = (jnp.dot(zs_r[...], wp1a_r[...],
                      preferred_element_type=jnp.float32)
              + jnp.dot(zd_r[...], wp1b_r[...],
                        preferred_element_type=jnp.float32)
              + jnp.dot(tf, wp1c_r[...], preferred_element_type=jnp.float32)
              + bp1_r[...])
        h1 = jnp.maximum(h1, 0.0)
        h2 = jnp.maximum(jnp.dot(h1, wp2_r[...],
                                 preferred_element_type=jnp.float32)
                         + bp2_r[...], 0.0)
        out_o[...] = jnp.dot(h2, wp3_r[...],
                             preferred_element_type=jnp.float32) + bp3_r[...]

    return pl.pallas_call(
        body,
        grid=(e // _BE,),
        in_specs=[_rows((_BE, d)), _rows((_BE, d)), _rows((_BE, 1)),
                  _full((1, 32)), _full((1, 32)), _full((32, t)),
                  _full((1, t)),
                  _full((d, 64)), _full((d, 64)), _full((t, 64)),
                  _full((1, 64)), _full((64, 32)), _full((1, 32)),
                  _full((32, 1)), _full((1, 1))],
        out_specs=_rows((_BE, 1)),
        out_shape=jax.ShapeDtypeStruct((e, 1), jnp.float32),
    )(zs, zd, ts, wt1, bt1, wt2, bt2,
      wp1a, wp1b, wp1c, bp1, wp2, bp2, wp3, bp3)


def kernel(x, edge_index, pred_edges, timestamps, W1_l, b1_l, W1_r,
           W2_l, b2_l, W2_r, Wt1, bt1, Wt2, bt2,
           Wp1, bp1, Wp2, bp2, Wp3, bp3):
    n, f_in = x.shape
    e = edge_index.shape[1]
    h = W1_l.shape[1]
    d = W2_l.shape[1]
    t = Wt2.shape[1]

    e_per_w = e // _NW
    nchunks = e_per_w // _B

    src3d = edge_index[0].reshape(_NW, nchunks, _B)
    dst3d = edge_index[1].reshape(_NW, nchunks, _B)
    ps3d = pred_edges[0].reshape(_NW, nchunks, _B)
    pd3d = pred_edges[1].reshape(_NW, nchunks, _B)

    zeros_h = jnp.zeros((n, h), jnp.float32)
    zeros_d = jnp.zeros((n, d), jnp.float32)
    zeros_dw = jnp.zeros((n, _DW), jnp.float32)
    ones_b = jnp.ones((_B, _DW), jnp.float32)

    xl, xr = _tc_in_proj(x, W1_l, W1_r)
    acc1, degp = _sc_segsum(xl, src3d, dst3d, zeros_h, zeros_dw, ones_b,
                            with_deg=True)
    hl, hr, deg = _tc_layer1(acc1[0], acc1[1], degp[0], degp[1],
                             b1_l.reshape(1, h), xr, W2_l, W2_r)

    (acc2,) = _sc_segsum(hl, src3d, dst3d, zeros_d, zeros_dw, ones_b,
                         with_deg=False)
    z = _tc_layer2(acc2[0], acc2[1], deg, b2_l.reshape(1, d), hr)

    zs, zd = _sc_gather(z, ps3d, pd3d)
    out = _tc_predict(zs, zd, timestamps.reshape(e, 1),
                      Wt1.reshape(1, 32), bt1.reshape(1, 32),
                      Wt2, bt2.reshape(1, t),
                      Wp1[:d], Wp1[d:2 * d], Wp1[2 * d:],
                      bp1.reshape(1, 64), Wp2, bp2.reshape(1, 32),
                      Wp3, bp3.reshape(1, 1))
    return out.reshape(e)


# speedup vs baseline: 1.3737x; 1.3737x over previous
import functools

import jax
import jax.numpy as jnp
from jax import lax
from jax.experimental import pallas as pl
from jax.experimental.pallas import tpu as pltpu
from jax.experimental.pallas import tpu_sc as plsc

_NC = 2
_NS = 16
_NW = _NC * _NS

_B = 80
_DW = 16


def _sc_segsum(rows, src3d, dst3d, zeros_rows, zeros_deg, ones_b, with_deg):
    n, w = rows.shape
    nchunks = src3d.shape[1]
    sub_stride = 8 * (n // (8 * _NS))
    sub_rows = n - sub_stride * (_NS - 1)

    mesh = plsc.VectorSubcoreMesh(core_axis_name="c", subcore_axis_name="s")

    out_type = [jax.ShapeDtypeStruct((_NC, n, w), jnp.float32)]
    if with_deg:
        out_type.append(jax.ShapeDtypeStruct((_NC, n, _DW), jnp.float32))

    scratch = [
        pltpu.VMEM((nchunks, _B), jnp.int32),
        pltpu.VMEM((nchunks, _B), jnp.int32),
        pltpu.VMEM((2, _B, w), jnp.float32),
        pltpu.VMEM((_B, _DW), jnp.float32),
        pltpu.SemaphoreType.DMA((2,)),
        pltpu.VMEM_SHARED((n, w), jnp.float32),
        pltpu.VMEM_SHARED((n, _DW), jnp.float32),
    ]

    def body(rows_hbm, src_hbm, dst_hbm, zr_hbm, zd_hbm, ones_hbm,
             *refs):
        if with_deg:
            acc_out, deg_out = refs[0], refs[1]
            scr = refs[2:]
        else:
            acc_out = refs[0]
            scr = refs[1:]
        src_v, dst_v, rows_v, ones_v, sem, acc_sh, deg_sh = scr

        c = lax.axis_index("c")
        s = lax.axis_index("s")
        wid = s * _NC + c

        r0 = s * sub_stride
        pltpu.sync_copy(zr_hbm.at[pl.ds(r0, sub_rows)],
                        acc_sh.at[pl.ds(r0, sub_rows)])
        if with_deg:
            pltpu.sync_copy(zd_hbm.at[pl.ds(r0, sub_rows)],
                            deg_sh.at[pl.ds(r0, sub_rows)])
            pltpu.sync_copy(ones_hbm, ones_v)

        pltpu.sync_copy(src_hbm.at[wid], src_v)
        pltpu.sync_copy(dst_hbm.at[wid], dst_v)
        pltpu.async_copy(rows_hbm.at[src_v.at[0]], rows_v.at[0], sem.at[0])
        plsc.subcore_barrier()

        def step(j, carry):
            slot = lax.rem(j, 2)
            nslot = lax.rem(j + 1, 2)

            @pl.when(j + 1 < nchunks)
            def _():
                pltpu.async_copy(rows_hbm.at[src_v.at[j + 1]],
                                 rows_v.at[nslot], sem.at[nslot])

            pltpu.make_async_copy(rows_hbm.at[src_v.at[j]],
                                  rows_v.at[slot], sem.at[slot]).wait()
            pltpu.sync_copy(rows_v.at[slot], acc_sh.at[dst_v.at[j]], add=True)
            if with_deg:
                pltpu.sync_copy(ones_v, deg_sh.at[dst_v.at[j]], add=True)
            return carry

        lax.fori_loop(0, nchunks, step, 0)
        plsc.subcore_barrier()

        pltpu.sync_copy(acc_sh.at[pl.ds(r0, sub_rows)],
                        acc_out.at[c].at[pl.ds(r0, sub_rows)])
        if with_deg:
            pltpu.sync_copy(deg_sh.at[pl.ds(r0, sub_rows)],
                            deg_out.at[c].at[pl.ds(r0, sub_rows)])

    k = pl.kernel(body, out_type=tuple(out_type), mesh=mesh,
                  scratch_types=scratch,
                  compiler_params=pltpu.CompilerParams(
                      use_tc_tiling_on_sc=False))
    return k(rows, src3d, dst3d, zeros_rows, zeros_deg, ones_b)


_BG = 64


def _sc_gather(z, ps3d, pd3d):
    n, w = z.shape
    nchunks = ps3d.shape[1]
    e_pad = _NW * nchunks * _BG

    mesh = plsc.VectorSubcoreMesh(core_axis_name="c", subcore_axis_name="s")

    out_type = jax.ShapeDtypeStruct((e_pad, 128), jnp.float32)
    scratch = [
        pltpu.VMEM((nchunks, _BG), jnp.int32),
        pltpu.VMEM((nchunks, _BG), jnp.int32),
        pltpu.VMEM((2, _BG, w), jnp.float32),
        pltpu.VMEM((2, _BG, w), jnp.float32),
        pltpu.SemaphoreType.DMA((2,)),
        pltpu.SemaphoreType.DMA((2,)),
    ]

    def body(z_hbm, ps_hbm, pd_hbm, zc_out,
             ps_v, pd_v, rs_v, rd_v, sem_s, sem_d):
        c = lax.axis_index("c")
        s = lax.axis_index("s")
        wid = s * _NC + c
        base = wid * nchunks * _BG

        pltpu.sync_copy(ps_hbm.at[wid], ps_v)
        pltpu.sync_copy(pd_hbm.at[wid], pd_v)
        pltpu.async_copy(z_hbm.at[ps_v.at[0]], rs_v.at[0], sem_s.at[0])
        pltpu.async_copy(z_hbm.at[pd_v.at[0]], rd_v.at[0], sem_d.at[0])

        def step(j, carry):
            slot = lax.rem(j, 2)
            nslot = lax.rem(j + 1, 2)

            @pl.when(j + 1 < nchunks)
            def _():
                pltpu.async_copy(z_hbm.at[ps_v.at[j + 1]],
                                 rs_v.at[nslot], sem_s.at[nslot])
                pltpu.async_copy(z_hbm.at[pd_v.at[j + 1]],
                                 rd_v.at[nslot], sem_d.at[nslot])

            row = base + j * _BG
            pltpu.make_async_copy(z_hbm.at[ps_v.at[j]],
                                  rs_v.at[slot], sem_s.at[slot]).wait()
            pltpu.sync_copy(rs_v.at[slot],
                            zc_out.at[pl.ds(row, _BG), pl.ds(0, w)])
            pltpu.make_async_copy(z_hbm.at[pd_v.at[j]],
                                  rd_v.at[slot], sem_d.at[slot]).wait()
            pltpu.sync_copy(rd_v.at[slot],
                            zc_out.at[pl.ds(row, _BG), pl.ds(w, w)])
            return carry

        lax.fori_loop(0, nchunks, step, 0)

    k = pl.kernel(body, out_type=out_type, mesh=mesh, scratch_types=scratch,
                  compiler_params=pltpu.CompilerParams(
                      use_tc_tiling_on_sc=False))
    return k(z, ps3d, pd3d)


_BN = 2000
_BE = 2000


def _full(shape):
    return pl.BlockSpec(shape, lambda i: tuple(0 for _ in shape))


def _rows(shape):
    return pl.BlockSpec(shape, lambda i: (i,) + tuple(0 for _ in shape[1:]))


def _tc_in_proj(x, wl, wr):
    n, f = x.shape
    h = wl.shape[1]

    def body(x_ref, wl_ref, wr_ref, xl_ref, xr_ref):
        xb = x_ref[...]
        xl_ref[...] = jnp.dot(xb, wl_ref[...],
                              preferred_element_type=jnp.float32)
        xr_ref[...] = jnp.dot(xb, wr_ref[...],
                              preferred_element_type=jnp.float32)

    return pl.pallas_call(
        body,
        grid=(n // _BN,),
        in_specs=[_rows((_BN, f)), _full((f, h)), _full((f, h))],
        out_specs=[_rows((_BN, h)), _rows((_BN, h))],
        out_shape=(jax.ShapeDtypeStruct((n, h), jnp.float32),
                   jax.ShapeDtypeStruct((n, h), jnp.float32)),
    )(x, wl, wr)


def _tc_layer1(a0, a1, d0, d1, b1, xr, w2l, w2r):
    n, h = a0.shape
    d = w2l.shape[1]

    def body(a0_r, a1_r, d0_r, d1_r, b1_r, xr_r, w2l_r, w2r_r,
             hl_o, hr_o, deg_o):
        deg = jnp.maximum(d0_r[...][:, 0:1] + d1_r[...][:, 0:1], 1.0)
        mean = (a0_r[...] + a1_r[...]) / deg
        hcur = jnp.maximum(mean + b1_r[...] + xr_r[...], 0.0)
        hl_o[...] = jnp.dot(hcur, w2l_r[...],
                            preferred_element_type=jnp.float32)
        hr_o[...] = jnp.dot(hcur, w2r_r[...],
                            preferred_element_type=jnp.float32)
        deg_o[...] = deg

    return pl.pallas_call(
        body,
        grid=(n // _BN,),
        in_specs=[_rows((_BN, h)), _rows((_BN, h)),
                  _rows((_BN, _DW)), _rows((_BN, _DW)),
                  _full((1, h)), _rows((_BN, h)),
                  _full((h, d)), _full((h, d))],
        out_specs=[_rows((_BN, d)), _rows((_BN, d)), _rows((_BN, 1))],
        out_shape=(jax.ShapeDtypeStruct((n, d), jnp.float32),
                   jax.ShapeDtypeStruct((n, d), jnp.float32),
                   jax.ShapeDtypeStruct((n, 1), jnp.float32)),
    )(a0, a1, d0, d1, b1, xr, w2l, w2r)


def _tc_layer2(a0, a1, deg, b2, hr):
    n, d = a0.shape

    def body(a0_r, a1_r, deg_r, b2_r, hr_r, z_o):
        z_o[...] = (a0_r[...] + a1_r[...]) / deg_r[...] + b2_r[...] + hr_r[...]

    return pl.pallas_call(
        body,
        grid=(n // _BN,),
        in_specs=[_rows((_BN, d)), _rows((_BN, d)), _rows((_BN, 1)),
                  _full((1, d)), _rows((_BN, d))],
        out_specs=_rows((_BN, d)),
        out_shape=jax.ShapeDtypeStruct((n, d), jnp.float32),
    )(a0, a1, deg, b2, hr)


_RB = 1024
_NR = 8


def _tc_predict(zc, tsw, wt1, bt1, wt2, bt2, wp1ab, wp1c, bp1, wp2, bp2,
                wp3, bp3):
    nrow = tsw.shape[0]
    grid = nrow // _NR

    def body(zc_ref, ts_ref, wt1_r, bt1_r, wt2_r, bt2_r, wp1_r, wp1c_r,
             bp1_r, wp2_r, bp2_r, wp3_r, bp3_r, out_ref):
        w25 = jnp.dot(wt2_r[...], wp1c_r[...],
                      preferred_element_type=jnp.float32)
        b25 = jnp.dot(bt2_r[...], wp1c_r[...],
                      preferred_element_type=jnp.float32)
        tcols = jnp.transpose(ts_ref[...], (1, 0))
        scores = []
        for r in range(_NR):
            zcr = zc_ref[r * _RB:(r + 1) * _RB, 0:64]
            tcol = tcols[:, r:r + 1]
            he = jnp.maximum(tcol * wt1_r[...] + bt1_r[...], 0.0)
            tw = jnp.dot(he, w25, preferred_element_type=jnp.float32) + b25
            h1 = jnp.maximum(
                jnp.dot(zcr, wp1_r[...],
                        preferred_element_type=jnp.float32) + tw + bp1_r[...],
                0.0)
            h2 = jnp.maximum(
                jnp.dot(h1, wp2_r[...],
                        preferred_element_type=jnp.float32) + bp2_r[...], 0.0)
            s = jnp.dot(h2, wp3_r[...],
                        preferred_element_type=jnp.float32) + bp3_r[...]
            scores.append(s)
        out_ref[...] = jnp.transpose(jnp.concatenate(scores, axis=1), (1, 0))

    return pl.pallas_call(
        body,
        grid=(grid,),
        in_specs=[pl.BlockSpec((_NR * _RB, 128), lambda i: (i, 0)),
                  pl.BlockSpec((_NR, _RB), lambda i: (i, 0)),
                  _full((1, 32)), _full((1, 32)), _full((32, 16)),
                  _full((1, 16)), _full((64, 64)), _full((16, 64)),
                  _full((1, 64)), _full((64, 32)), _full((1, 32)),
                  _full((32, 1)), _full((1, 1))],
        out_specs=pl.BlockSpec((_NR, _RB), lambda i: (i, 0)),
        out_shape=jax.ShapeDtypeStruct((nrow, _RB), jnp.float32),
    )(zc, tsw, wt1, bt1, wt2, bt2, wp1ab, wp1c, bp1, wp2, bp2, wp3, bp3)


def kernel(x, edge_index, pred_edges, timestamps, W1_l, b1_l, W1_r,
           W2_l, b2_l, W2_r, Wt1, bt1, Wt2, bt2,
           Wp1, bp1, Wp2, bp2, Wp3, bp3):
    n, f_in = x.shape
    e = edge_index.shape[1]
    h = W1_l.shape[1]
    d = W2_l.shape[1]
    t = Wt2.shape[1]

    e_per_w = e // _NW
    nchunks = e_per_w // _B

    src3d = edge_index[0].reshape(_NW, nchunks, _B)
    dst3d = edge_index[1].reshape(_NW, nchunks, _B)

    tile = _NR * _RB
    e_pad = -(-e // tile) * tile
    nch_g = e_pad // _NW // _BG
    pad = e_pad - e
    pad0 = jnp.zeros((pad,), jnp.int32)
    ps3d = jnp.concatenate([pred_edges[0], pad0]).reshape(_NW, nch_g, _BG)
    pd3d = jnp.concatenate([pred_edges[1], pad0]).reshape(_NW, nch_g, _BG)
    tsw = jnp.concatenate([timestamps, jnp.zeros((pad,), jnp.float32)]
                          ).reshape(e_pad // _RB, _RB)

    zeros_h = jnp.zeros((n, h), jnp.float32)
    zeros_d = jnp.zeros((n, d), jnp.float32)
    zeros_dw = jnp.zeros((n, _DW), jnp.float32)
    ones_b = jnp.ones((_B, _DW), jnp.float32)

    xl, xr = _tc_in_proj(x, W1_l, W1_r)
    acc1, degp = _sc_segsum(xl, src3d, dst3d, zeros_h, zeros_dw, ones_b,
                            with_deg=True)
    hl, hr, deg = _tc_layer1(acc1[0], acc1[1], degp[0], degp[1],
                             b1_l.reshape(1, h), xr, W2_l, W2_r)

    (acc2,) = _sc_segsum(hl, src3d, dst3d, zeros_d, zeros_dw, ones_b,
                         with_deg=False)
    z = _tc_layer2(acc2[0], acc2[1], deg, b2_l.reshape(1, d), hr)

    zc = _sc_gather(z, ps3d, pd3d)
    out = _tc_predict(zc, tsw,
                      Wt1.reshape(1, 32), bt1.reshape(1, 32),
                      Wt2, bt2.reshape(1, t),
                      Wp1[:2 * d], Wp1[2 * d:],
                      bp1.reshape(1, 64), Wp2, bp2.reshape(1, 32),
                      Wp3, bp3.reshape(1, 1))
    return out.reshape(e_pad)[:e]
